# baseline (speedup 1.0000x reference)
import functools

import jax
import jax.numpy as jnp
from jax import lax
from jax.experimental import pallas as pl
from jax.experimental.pallas import tpu as pltpu
from jax.experimental.pallas import tpu_sc as plsc

_GROUPS = [(0, 'hp'), (1, 'stat'), (2, 'stat'), (3, 'stat'), (4, 'stat'),
           (5, 'stat'), (6, 'stat'), (7, 'power'), (8, 'power'), (9, 'power'),
           (10, 'power'), (11, 'damage'), (12, 'damage'), (13, 'damage'),
           (14, 'damage'), (15, 'turn'), (16, 'rating'), (17, 'rating')]
_CFG = {'hp': (1.0, 100, 16), 'stat': (600.0, 600, 16), 'power': (250.0, 250, 16),
        'damage': (600.0, 600, 4), 'turn': (40.0, 40, 16), 'rating': (2000.0, 100, 16)}
_BANK_ORDER = ['hp', 'stat', 'power', 'damage', 'turn', 'rating']

_BASES = {}
_off = 0
for _name in _BANK_ORDER:
    _BASES[_name] = _off
    _maxv, _nbins, _w = _CFG[_name]
    _off += (_nbins + 1) * (_w + 1)
_BANK_WORDS = _off
_BANK_PAD = (-_BANK_WORDS) % 16
_BANK_TOTAL = _BANK_WORDS + _BANK_PAD

_FEATS = []
for _xcol, _name in _GROUPS:
    _maxv, _nbins, _w = _CFG[_name]
    _FEATS.append((_xcol, _BASES[_name], _w + 1, _w, _maxv, _nbins))

_COLPLAN = []
for _fi, (_xcol, _b, _stride, _w, _mv, _nb) in enumerate(_FEATS):
    for _o in range(_w):
        _COLPLAN.append((_fi, _o))
_N_EMB = len(_COLPLAN)

_D_IN = 64
_D_OUT = _N_EMB + (_D_IN - len(_FEATS))
_NC, _NS = 2, 16
_NW = _NC * _NS
_CHB = 4
_L = 16


def _splat(v):
    return jnp.full((_L,), v, jnp.int32)


def _tec_body(x_hbm, banks_hbm, out_hbm, xv, banksv, outv, sem_in, sem_out, *, bsz, seq):
    rows_per = bsz // _NW
    n_ch = rows_per // _CHB
    pos_per_ch = _CHB * seq
    n_groups = (pos_per_ch + _L - 1) // _L
    wid = lax.axis_index("s") * _NC + lax.axis_index("c")
    base_b = wid * rows_per

    pltpu.sync_copy(banks_hbm, banksv)

    iota = lax.iota(jnp.int32, _L)

    def chunk_body(g, carry):
        b0 = base_b + g * _CHB
        cps = [pltpu.async_copy(x_hbm.at[b0 + r], xv.at[pl.ds(r * seq, seq)], sem_in)
               for r in range(_CHB)]
        for cp in cps:
            cp.wait()

        def do_group(pv, mask):
            rowaddr = []
            for (xcol, bank_base, stride, w, maxv, nbins) in _FEATS:
                raw = plsc.load_gather(xv, [pv, _splat(xcol)])
                clamped = jnp.clip(raw, 0.0, maxv)
                b = ((clamped / maxv) * nbins).astype(jnp.int32)
                b = jnp.clip(b, 0, nbins)
                rowaddr.append(bank_base + b * stride)

            def col_val(col):
                if col < _N_EMB:
                    fi, off = _COLPLAN[col]
                    return plsc.load_gather(banksv, [rowaddr[fi] + off])
                return plsc.load_gather(xv, [pv, _splat(col - _N_EMB + len(_FEATS))])

            _K = 8
            for lo in range(0, _D_OUT, _K):
                batch = range(lo, min(lo + _K, _D_OUT))
                vals = [col_val(col) for col in batch]
                for col, val in zip(batch, vals):
                    plsc.store_scatter(outv, [pv, _splat(col)], val, mask=mask)

        n_full = pos_per_ch // _L

        @plsc.parallel_loop(0, n_full)
        def group_body(t):
            do_group(t * _L + iota, None)

        if pos_per_ch % _L:
            @plsc.parallel_loop(n_full, n_full + 1)
            def tail_body(t):
                pv_raw = t * _L + iota
                do_group(jnp.minimum(pv_raw, pos_per_ch - 1),
                         pv_raw < pos_per_ch)

        cps = [pltpu.async_copy(outv.at[pl.ds(r * seq, seq)], out_hbm.at[b0 + r], sem_out)
               for r in range(_CHB)]
        for cp in cps:
            cp.wait()
        return carry

    lax.fori_loop(0, n_ch, chunk_body, 0)


def kernel(x, hp_bank, stat_bank, power_bank, damage_bank, turn_bank,
           rating_bank, group_idx):
    bsz, seq, d_in = x.shape

    def _padrow(b):
        return jnp.pad(b, ((0, 0), (0, 1))).reshape(-1)

    banks_flat = jnp.concatenate([
        _padrow(hp_bank), _padrow(stat_bank), _padrow(power_bank),
        _padrow(damage_bank), _padrow(turn_bank), _padrow(rating_bank),
        jnp.zeros((_BANK_PAD,), jnp.float32)])

    mesh = plsc.VectorSubcoreMesh(core_axis_name="c", subcore_axis_name="s")
    run = functools.partial(
        pl.kernel,
        mesh=mesh,
        compiler_params=pltpu.CompilerParams(
            needs_layout_passes=False, use_tc_tiling_on_sc=False,
            disable_bounds_checks=True),
        out_type=jax.ShapeDtypeStruct((bsz, seq, _D_OUT), jnp.float32),
        scratch_types=[
            pltpu.VMEM((_CHB * seq, _D_IN), jnp.float32),
            pltpu.VMEM((_BANK_TOTAL,), jnp.float32),
            pltpu.VMEM((_CHB * seq, _D_OUT), jnp.float32),
            pltpu.SemaphoreType.DMA,
            pltpu.SemaphoreType.DMA,
        ],
    )(functools.partial(_tec_body, bsz=bsz, seq=seq))
    return run(x, banks_flat)


# speedup vs baseline: 1.4025x; 1.4025x over previous
import functools

import jax
import jax.numpy as jnp
from jax import lax
from jax.experimental import pallas as pl
from jax.experimental.pallas import tpu as pltpu
from jax.experimental.pallas import tpu_sc as plsc

_GROUPS = [(0, 'hp'), (1, 'stat'), (2, 'stat'), (3, 'stat'), (4, 'stat'),
           (5, 'stat'), (6, 'stat'), (7, 'power'), (8, 'power'), (9, 'power'),
           (10, 'power'), (11, 'damage'), (12, 'damage'), (13, 'damage'),
           (14, 'damage'), (15, 'turn'), (16, 'rating'), (17, 'rating')]
_CFG = {'hp': (1.0, 100, 16), 'stat': (600.0, 600, 16), 'power': (250.0, 250, 16),
        'damage': (600.0, 600, 4), 'turn': (40.0, 40, 16), 'rating': (2000.0, 100, 16)}
_BANK_ORDER = ['hp', 'stat', 'power', 'damage', 'turn', 'rating']

_BASES = {}
_off = 0
for _name in _BANK_ORDER:
    _BASES[_name] = _off
    _maxv, _nbins, _w = _CFG[_name]
    _off += (_nbins + 1) * (_w + 1)
_BANK_WORDS = _off
_BANK_PAD = (-_BANK_WORDS) % 16
_BANK_TOTAL = _BANK_WORDS + _BANK_PAD

_FEATS = []
for _xcol, _name in _GROUPS:
    _maxv, _nbins, _w = _CFG[_name]
    _FEATS.append((_xcol, _BASES[_name], _w + 1, _w, _maxv, _nbins))

_COLPLAN = []
for _fi, (_xcol, _b, _stride, _w, _mv, _nb) in enumerate(_FEATS):
    for _o in range(_w):
        _COLPLAN.append((_fi, _o))
_N_EMB = len(_COLPLAN)

_D_IN = 64
_D_OUT = _N_EMB + (_D_IN - len(_FEATS))
_NC, _NS = 2, 16
_NW = _NC * _NS
_CH = 128
_L = 16


def _tec_body(x_hbm, banks_hbm, out_hbm,
              xv0, xv1, banksv, ov0, ov1, si0, si1, so0, so1, *, n_pos):
    p_per = n_pos // _NW
    n_ch = p_per // _CH
    wid = lax.axis_index("s") * _NC + lax.axis_index("c")
    base = wid * p_per

    xvs, ovs, sis, sos = (xv0, xv1), (ov0, ov1), (si0, si1), (so0, so1)

    pltpu.sync_copy(banks_hbm, banksv)

    iota = lax.iota(jnp.int32, _L)
    iota_in = iota * _D_IN
    iota_out = iota * _D_OUT

    def start_in(chunk, b):
        row0 = base + chunk * _CH
        pltpu.async_copy(x_hbm.at[pl.ds(row0 * _D_IN, _CH * _D_IN)],
                         xvs[b], sis[b])

    def wait_in(b):
        pltpu.make_async_copy(x_hbm.at[pl.ds(0, _CH * _D_IN)],
                              xvs[b], sis[b]).wait()

    def start_out(chunk, b):
        row0 = base + chunk * _CH
        pltpu.async_copy(ovs[b],
                         out_hbm.at[pl.ds(row0 * _D_OUT, _CH * _D_OUT)],
                         sos[b])

    def wait_out(b):
        pltpu.make_async_copy(ovs[b],
                              out_hbm.at[pl.ds(0, _CH * _D_OUT)],
                              sos[b]).wait()

    def compute(b):
        xv, outv = xvs[b], ovs[b]

        @plsc.parallel_loop(0, _CH // _L)
        def group_body(t):
            pb = iota_in + t * (_L * _D_IN)
            ob = iota_out + t * (_L * _D_OUT)
            rowaddr = []
            for (xcol, bank_base, stride, w, maxv, nbins) in _FEATS:
                raw = plsc.load_gather(xv, [pb + xcol])
                clamped = jnp.clip(raw, 0.0, maxv)
                bk = ((clamped / maxv) * nbins).astype(jnp.int32)
                bk = jnp.clip(bk, 0, nbins)
                rowaddr.append(bank_base + bk * stride)

            def col_val(col):
                if col < _N_EMB:
                    fi, off = _COLPLAN[col]
                    return plsc.load_gather(banksv, [rowaddr[fi] + off])
                return plsc.load_gather(xv, [pb + (col - _N_EMB + len(_FEATS))])

            _K = 8
            for lo in range(0, _D_OUT, _K):
                batch = range(lo, min(lo + _K, _D_OUT))
                vals = [col_val(col) for col in batch]
                for col, val in zip(batch, vals):
                    plsc.store_scatter(outv, [ob + col], val)

    start_in(0, 0)

    def pair_body(p, carry):
        for b in range(2):
            chunk = p * 2 + b

            @pl.when(chunk + 1 < n_ch)
            def _():
                start_in(chunk + 1, 1 - b)

            wait_in(b)

            @pl.when(chunk >= 2)
            def _():
                wait_out(b)

            compute(b)
            start_out(chunk, b)
        return carry

    lax.fori_loop(0, n_ch // 2, pair_body, 0)
    wait_out(0)
    wait_out(1)


def kernel(x, hp_bank, stat_bank, power_bank, damage_bank, turn_bank,
           rating_bank, group_idx):
    bsz, seq, d_in = x.shape
    n_pos = bsz * seq

    def _padrow(b):
        return jnp.pad(b, ((0, 0), (0, 1))).reshape(-1)

    banks_flat = jnp.concatenate([
        _padrow(hp_bank), _padrow(stat_bank), _padrow(power_bank),
        _padrow(damage_bank), _padrow(turn_bank), _padrow(rating_bank),
        jnp.zeros((_BANK_PAD,), jnp.float32)])

    mesh = plsc.VectorSubcoreMesh(core_axis_name="c", subcore_axis_name="s")
    run = functools.partial(
        pl.kernel,
        mesh=mesh,
        compiler_params=pltpu.CompilerParams(
            needs_layout_passes=False, use_tc_tiling_on_sc=False,
            disable_bounds_checks=True),
        out_type=jax.ShapeDtypeStruct((n_pos * _D_OUT,), jnp.float32),
        scratch_types=[
            pltpu.VMEM((_CH * _D_IN,), jnp.float32),
            pltpu.VMEM((_CH * _D_IN,), jnp.float32),
            pltpu.VMEM((_BANK_TOTAL,), jnp.float32),
            pltpu.VMEM((_CH * _D_OUT,), jnp.float32),
            pltpu.VMEM((_CH * _D_OUT,), jnp.float32),
            pltpu.SemaphoreType.DMA,
            pltpu.SemaphoreType.DMA,
            pltpu.SemaphoreType.DMA,
            pltpu.SemaphoreType.DMA,
        ],
    )(functools.partial(_tec_body, n_pos=n_pos))
    out_flat = run(x.reshape(-1), banks_flat)
    return out_flat.reshape(bsz, seq, _D_OUT)


# speedup vs baseline: 1.4896x; 1.0621x over previous
import functools

import jax
import jax.numpy as jnp
from jax import lax
from jax.experimental import pallas as pl
from jax.experimental.pallas import tpu as pltpu
from jax.experimental.pallas import tpu_sc as plsc

_GROUPS = [(0, 'hp'), (1, 'stat'), (2, 'stat'), (3, 'stat'), (4, 'stat'),
           (5, 'stat'), (6, 'stat'), (7, 'power'), (8, 'power'), (9, 'power'),
           (10, 'power'), (11, 'damage'), (12, 'damage'), (13, 'damage'),
           (14, 'damage'), (15, 'turn'), (16, 'rating'), (17, 'rating')]
_CFG = {'hp': (1.0, 100, 16), 'stat': (600.0, 600, 16), 'power': (250.0, 250, 16),
        'damage': (600.0, 600, 4), 'turn': (40.0, 40, 16), 'rating': (2000.0, 100, 16)}
_BANK_ORDER = ['hp', 'stat', 'power', 'damage', 'turn', 'rating']

_BASES = {}
_off = 0
for _name in _BANK_ORDER:
    _BASES[_name] = _off
    _maxv, _nbins, _w = _CFG[_name]
    _off += (_nbins + 1) * (_w + 1)
_BANK_WORDS = _off
_BANK_PAD = (-_BANK_WORDS) % 16
_BANK_TOTAL = _BANK_WORDS + _BANK_PAD

_FEATS = []
for _xcol, _name in _GROUPS:
    _maxv, _nbins, _w = _CFG[_name]
    _FEATS.append((_xcol, _BASES[_name], _w + 1, _w, _maxv, _nbins))

_COLPLAN = []
for _fi, (_xcol, _b, _stride, _w, _mv, _nb) in enumerate(_FEATS):
    for _o in range(_w):
        _COLPLAN.append((_fi, _o))
_N_EMB = len(_COLPLAN)

_D_IN = 64
_D_OUT = _N_EMB + (_D_IN - len(_FEATS))
_NC, _NS = 2, 16
_NW = _NC * _NS
_CH = 256
_HCH = _CH // 2
_L = 16


def _tec_body(x_hbm, banks_hbm, out_hbm,
              xv, banksv, ov0, ov1, so0, so1, *, n_pos):
    p_per = n_pos // _NW
    n_ch = p_per // _CH
    wid = lax.axis_index("s") * _NC + lax.axis_index("c")
    base = wid * p_per

    ovs, sos = (ov0, ov1), (so0, so1)

    pltpu.sync_copy(banks_hbm, banksv)
    pltpu.async_copy(ov0, out_hbm.at[pl.ds(base * _D_OUT, _HCH * _D_OUT)], so0)
    pltpu.async_copy(ov1, out_hbm.at[pl.ds((base + _HCH) * _D_OUT,
                                           _HCH * _D_OUT)], so1)

    iota = lax.iota(jnp.int32, _L)
    iota_in = iota * _D_IN
    iota_out = iota * _D_OUT

    def wait_out(h):
        pltpu.make_async_copy(ovs[h],
                              out_hbm.at[pl.ds(0, _HCH * _D_OUT)],
                              sos[h]).wait()

    def chunk_body(g, carry):
        row0 = base + g * _CH
        pltpu.sync_copy(x_hbm.at[pl.ds(row0 * _D_IN, _CH * _D_IN)], xv)

        for h in range(2):
            wait_out(h)
            outv = ovs[h]

            @plsc.parallel_loop(0, _HCH // _L)
            def group_body(t):
                th = t + h * (_HCH // _L)
                pb = iota_in + th * (_L * _D_IN)
                ob = iota_out + t * (_L * _D_OUT)
                rowaddr = []
                for (xcol, bank_base, stride, w, maxv, nbins) in _FEATS:
                    raw = plsc.load_gather(xv, [pb + xcol])
                    clamped = jnp.clip(raw, 0.0, maxv)
                    bk = ((clamped / maxv) * nbins).astype(jnp.int32)
                    bk = jnp.clip(bk, 0, nbins)
                    rowaddr.append(bank_base + bk * stride)

                def col_val(col):
                    if col < _N_EMB:
                        fi, off = _COLPLAN[col]
                        return plsc.load_gather(banksv, [rowaddr[fi] + off])
                    return plsc.load_gather(
                        xv, [pb + (col - _N_EMB + len(_FEATS))])

                _K = 8
                for lo in range(0, _D_OUT, _K):
                    batch = range(lo, min(lo + _K, _D_OUT))
                    vals = [col_val(col) for col in batch]
                    for col, val in zip(batch, vals):
                        plsc.store_scatter(outv, [ob + col], val)

            pltpu.async_copy(
                outv,
                out_hbm.at[pl.ds((row0 + h * _HCH) * _D_OUT, _HCH * _D_OUT)],
                sos[h])
        return carry

    lax.fori_loop(0, n_ch, chunk_body, 0)
    wait_out(0)
    wait_out(1)


def kernel(x, hp_bank, stat_bank, power_bank, damage_bank, turn_bank,
           rating_bank, group_idx):
    bsz, seq, d_in = x.shape
    n_pos = bsz * seq

    def _padrow(b):
        return jnp.pad(b, ((0, 0), (0, 1))).reshape(-1)

    banks_flat = jnp.concatenate([
        _padrow(hp_bank), _padrow(stat_bank), _padrow(power_bank),
        _padrow(damage_bank), _padrow(turn_bank), _padrow(rating_bank),
        jnp.zeros((_BANK_PAD,), jnp.float32)])

    mesh = plsc.VectorSubcoreMesh(core_axis_name="c", subcore_axis_name="s")
    run = functools.partial(
        pl.kernel,
        mesh=mesh,
        compiler_params=pltpu.CompilerParams(
            needs_layout_passes=False, use_tc_tiling_on_sc=False),
        out_type=jax.ShapeDtypeStruct((n_pos * _D_OUT,), jnp.float32),
        scratch_types=[
            pltpu.VMEM((_CH * _D_IN,), jnp.float32),
            pltpu.VMEM((_BANK_TOTAL,), jnp.float32),
            pltpu.VMEM((_HCH * _D_OUT,), jnp.float32),
            pltpu.VMEM((_HCH * _D_OUT,), jnp.float32),
            pltpu.SemaphoreType.DMA,
            pltpu.SemaphoreType.DMA,
        ],
    )(functools.partial(_tec_body, n_pos=n_pos))
    out_flat = run(x.reshape(-1), banks_flat)
    return out_flat.reshape(bsz, seq, _D_OUT)
